# Initial kernel scaffold; baseline (speedup 1.0000x reference)
import jax
import jax.numpy as jnp
from jax.experimental import pallas as pl


def kernel(x, edge_index, batch, W, w_ih, w_hh, b_ih, b_hh, fc1_w, fc1_b, bn1_g, bn1_b, fc2_w, fc2_b, bn2_g, bn2_b, fc3_w, fc3_b):
    raise NotImplementedError("write your pallas kernel here")


# speedup vs baseline: 7.4049x; 7.4049x over previous
import functools

import jax
import jax.numpy as jnp
from jax import lax
from jax.experimental import pallas as pl
from jax.experimental.pallas import tpu as pltpu
from jax.experimental.pallas import tpu_sc as plsc

N = 10000
E = 320000
D1 = 128
D2 = 64
D3 = 32
C = 10
G = 64

NC = 2
NS = 16
NW = NC * NS
K = 125
NCHUNK = E // K // NW
EPW = K * NCHUNK
NPAD = 10240
RPT = NPAD // NS

BN = 2000
NB = N // BN


def _sc_agg_body(m_hbm, src_hbm, dst_hbm, zD_hbm,
                 out_agg,
                 src_v, dst_v, rows_v, sh_agg, sem):
    cid = lax.axis_index("c")
    sid = lax.axis_index("s")
    base_n = sid * RPT
    pltpu.sync_copy(zD_hbm.at[pl.ds(base_n, RPT)],
                    sh_agg.at[pl.ds(base_n, RPT)])
    wid = cid * NS + sid
    pltpu.sync_copy(src_hbm.at[pl.ds(wid * NCHUNK, NCHUNK)], src_v)
    pltpu.sync_copy(dst_hbm.at[pl.ds(wid * NCHUNK, NCHUNK)], dst_v)
    plsc.subcore_barrier()

    def chunk(i, carry):
        pltpu.async_copy(m_hbm.at[src_v.at[i]], rows_v, sem).wait()
        pltpu.sync_copy(rows_v, sh_agg.at[dst_v.at[i]], add=True)
        return carry

    lax.fori_loop(0, NCHUNK, chunk, 0)
    plsc.subcore_barrier()
    base_o = cid * NPAD + base_n
    pltpu.sync_copy(sh_agg.at[pl.ds(base_n, RPT)],
                    out_agg.at[pl.ds(base_o, RPT)])


@functools.lru_cache(maxsize=None)
def _sc_agg():
    mesh = plsc.VectorSubcoreMesh(core_axis_name="c", subcore_axis_name="s")
    return pl.kernel(
        _sc_agg_body,
        mesh=mesh,
        out_type=jax.ShapeDtypeStruct((NC * NPAD, D1), jnp.float32),
        scratch_types=[
            pltpu.VMEM((NCHUNK, K), jnp.int32),
            pltpu.VMEM((NCHUNK, K), jnp.int32),
            pltpu.VMEM((K, D1), jnp.float32),
            pltpu.VMEM_SHARED((NPAD, D1), jnp.float32),
            pltpu.SemaphoreType.DMA,
        ],
    )


def _sc_aggregate(m, src2, dst2, zD):
    return _sc_agg()(m, src2, dst2, zD).reshape(NC, NPAD, D1)


def _sc_deg_body(dst_hbm, zD_hbm, ones_hbm,
                 out_deg,
                 dst_v, ones_v, sh_deg, sem):
    cid = lax.axis_index("c")
    sid = lax.axis_index("s")
    base_n = sid * RPT
    pltpu.sync_copy(zD_hbm.at[pl.ds(base_n, RPT)],
                    sh_deg.at[pl.ds(base_n, RPT)])
    pltpu.sync_copy(ones_hbm, ones_v)
    wid = cid * NS + sid
    pltpu.sync_copy(dst_hbm.at[pl.ds(wid * NCHUNK, NCHUNK)], dst_v)
    plsc.subcore_barrier()

    def chunk(i, carry):
        pltpu.sync_copy(ones_v, sh_deg.at[dst_v.at[i]], add=True)
        return carry

    lax.fori_loop(0, NCHUNK, chunk, 0)
    plsc.subcore_barrier()
    base_o = cid * NPAD + base_n
    pltpu.sync_copy(sh_deg.at[pl.ds(base_n, RPT)],
                    out_deg.at[pl.ds(base_o, RPT)])


@functools.lru_cache(maxsize=None)
def _sc_deg():
    mesh = plsc.VectorSubcoreMesh(core_axis_name="c", subcore_axis_name="s")
    return pl.kernel(
        _sc_deg_body,
        mesh=mesh,
        out_type=jax.ShapeDtypeStruct((NC * NPAD, D1), jnp.float32),
        scratch_types=[
            pltpu.VMEM((NCHUNK, K), jnp.int32),
            pltpu.VMEM((K, D1), jnp.float32),
            pltpu.VMEM_SHARED((NPAD, D1), jnp.float32),
            pltpu.SemaphoreType.DMA,
        ],
    )


def _sc_degree(dst2, zD, ones):
    return _sc_deg()(dst2, zD, ones).reshape(NC, NPAD, D1)


def _mm_body(h_ref, w_ref, o_ref):
    o_ref[...] = jnp.dot(h_ref[...], w_ref[...],
                         preferred_element_type=jnp.float32)


def _mm(h, w):
    return pl.pallas_call(
        _mm_body,
        grid=(NB,),
        in_specs=[pl.BlockSpec((BN, D1), lambda i: (i, 0)),
                  pl.BlockSpec((D1, D1), lambda i: (0, 0))],
        out_specs=pl.BlockSpec((BN, D1), lambda i: (i, 0)),
        out_shape=jax.ShapeDtypeStruct((N, D1), jnp.float32),
    )(h, w)


def _gru_math(h, p0, p1, d0, d1, wih, whh, bih, bhh):
    deg = jnp.maximum(d0[:, :1] + d1[:, :1], 1.0)
    agg = (p0 + p1) / deg
    gi = jnp.dot(agg, wih, preferred_element_type=jnp.float32) + bih
    gh = jnp.dot(h, whh, preferred_element_type=jnp.float32) + bhh
    r = jax.nn.sigmoid(gi[:, :D1] + gh[:, :D1])
    z = jax.nn.sigmoid(gi[:, D1:2 * D1] + gh[:, D1:2 * D1])
    n = jnp.tanh(gi[:, 2 * D1:] + r * gh[:, 2 * D1:])
    return (1.0 - z) * n + z * h


def _gru_mid_body(h_ref, p0_ref, p1_ref, d0_ref, d1_ref,
                  wih_ref, whh_ref, bih_ref, bhh_ref, w2_ref,
                  h1_ref, m1_ref):
    h1 = _gru_math(h_ref[...], p0_ref[...], p1_ref[...],
                   d0_ref[...], d1_ref[...],
                   wih_ref[...], whh_ref[...], bih_ref[...], bhh_ref[...])
    h1_ref[...] = h1
    m1_ref[...] = jnp.dot(h1, w2_ref[...], preferred_element_type=jnp.float32)


def _gru_mid(h, p0, p1, d0, d1, wih, whh, bih, bhh, w2):
    return pl.pallas_call(
        _gru_mid_body,
        grid=(NB,),
        in_specs=[
            pl.BlockSpec((BN, D1), lambda i: (i, 0)),
            pl.BlockSpec((BN, D1), lambda i: (i, 0)),
            pl.BlockSpec((BN, D1), lambda i: (i, 0)),
            pl.BlockSpec((BN, D1), lambda i: (i, 0)),
            pl.BlockSpec((BN, D1), lambda i: (i, 0)),
            pl.BlockSpec((D1, 3 * D1), lambda i: (0, 0)),
            pl.BlockSpec((D1, 3 * D1), lambda i: (0, 0)),
            pl.BlockSpec((1, 3 * D1), lambda i: (0, 0)),
            pl.BlockSpec((1, 3 * D1), lambda i: (0, 0)),
            pl.BlockSpec((D1, D1), lambda i: (0, 0)),
        ],
        out_specs=[pl.BlockSpec((BN, D1), lambda i: (i, 0)),
                   pl.BlockSpec((BN, D1), lambda i: (i, 0))],
        out_shape=[jax.ShapeDtypeStruct((N, D1), jnp.float32),
                   jax.ShapeDtypeStruct((N, D1), jnp.float32)],
    )(h, p0, p1, d0, d1, wih, whh, bih, bhh, w2)


def _gru_pool_body(h_ref, p0_ref, p1_ref, d0_ref, d1_ref,
                   wih_ref, whh_ref, bih_ref, bhh_ref, b_ref,
                   pool_ref, cnt_ref):
    h2 = _gru_math(h_ref[...], p0_ref[...], p1_ref[...],
                   d0_ref[...], d1_ref[...],
                   wih_ref[...], whh_ref[...], bih_ref[...], bhh_ref[...])
    hr = jnp.maximum(h2, 0.0)
    b = b_ref[0, 0, :]
    oh = (b[:, None] == lax.broadcasted_iota(jnp.int32, (BN, G), 1)
          ).astype(jnp.float32)
    pool_part = lax.dot_general(oh, hr, (((0,), (0,)), ((), ())),
                                preferred_element_type=jnp.float32)
    cnt_part = jnp.sum(oh, axis=0)[:, None] * jnp.ones((1, D1), jnp.float32)

    @pl.when(pl.program_id(0) == 0)
    def _init():
        pool_ref[...] = jnp.zeros_like(pool_ref)
        cnt_ref[...] = jnp.zeros_like(cnt_ref)

    pool_ref[...] += pool_part
    cnt_ref[...] += cnt_part


def _gru_pool(h, p0, p1, d0, d1, wih, whh, bih, bhh, batch3):
    return pl.pallas_call(
        _gru_pool_body,
        grid=(NB,),
        in_specs=[
            pl.BlockSpec((BN, D1), lambda i: (i, 0)),
            pl.BlockSpec((BN, D1), lambda i: (i, 0)),
            pl.BlockSpec((BN, D1), lambda i: (i, 0)),
            pl.BlockSpec((BN, D1), lambda i: (i, 0)),
            pl.BlockSpec((BN, D1), lambda i: (i, 0)),
            pl.BlockSpec((D1, 3 * D1), lambda i: (0, 0)),
            pl.BlockSpec((D1, 3 * D1), lambda i: (0, 0)),
            pl.BlockSpec((1, 3 * D1), lambda i: (0, 0)),
            pl.BlockSpec((1, 3 * D1), lambda i: (0, 0)),
            pl.BlockSpec((1, 1, BN), lambda i: (i, 0, 0)),
        ],
        out_specs=[pl.BlockSpec((G, D1), lambda i: (0, 0)),
                   pl.BlockSpec((G, D1), lambda i: (0, 0))],
        out_shape=[jax.ShapeDtypeStruct((G, D1), jnp.float32),
                   jax.ShapeDtypeStruct((G, D1), jnp.float32)],
    )(h, p0, p1, d0, d1, wih, whh, bih, bhh, batch3)


def _bn_relu(y, g, b):
    mu = jnp.mean(y, axis=0, keepdims=True)
    var = jnp.mean((y - mu) * (y - mu), axis=0, keepdims=True)
    return jnp.maximum((y - mu) / jnp.sqrt(var + 1e-5) * g + b, 0.0)


def _mlp_body(pool_ref, cnt_ref,
              f1_ref, b1_ref, g1_ref, be1_ref,
              f2_ref, b2_ref, g2_ref, be2_ref,
              f3_ref, b3_ref, o_ref):
    p = pool_ref[...] / jnp.maximum(cnt_ref[...], 1.0)
    y = jnp.dot(p, f1_ref[...], preferred_element_type=jnp.float32) + b1_ref[...]
    y = _bn_relu(y, g1_ref[...], be1_ref[...])
    y = jnp.dot(y, f2_ref[...], preferred_element_type=jnp.float32) + b2_ref[...]
    y = _bn_relu(y, g2_ref[...], be2_ref[...])
    y = jnp.dot(y, f3_ref[...], preferred_element_type=jnp.float32) + b3_ref[...]
    m = jnp.max(y, axis=1, keepdims=True)
    lse = jnp.log(jnp.sum(jnp.exp(y - m), axis=1, keepdims=True)) + m
    o_ref[...] = y - lse


def _mlp(pool, cnt, f1t, b1, g1, be1, f2t, b2, g2, be2, f3t, b3):
    full = lambda i: (0, 0)
    return pl.pallas_call(
        _mlp_body,
        grid=(1,),
        in_specs=[
            pl.BlockSpec((G, D1), full), pl.BlockSpec((G, D1), full),
            pl.BlockSpec((D1, D2), full), pl.BlockSpec((1, D2), full),
            pl.BlockSpec((1, D2), full), pl.BlockSpec((1, D2), full),
            pl.BlockSpec((D2, D3), full), pl.BlockSpec((1, D3), full),
            pl.BlockSpec((1, D3), full), pl.BlockSpec((1, D3), full),
            pl.BlockSpec((D3, C), full), pl.BlockSpec((1, C), full),
        ],
        out_specs=pl.BlockSpec((G, C), full),
        out_shape=jax.ShapeDtypeStruct((G, C), jnp.float32),
    )(pool, cnt, f1t, b1, g1, be1, f2t, b2, g2, be2, f3t, b3)


def kernel(x, edge_index, batch, W, w_ih, w_hh, b_ih, b_hh,
           fc1_w, fc1_b, bn1_g, bn1_b, fc2_w, fc2_b, bn2_g, bn2_b,
           fc3_w, fc3_b):
    src = edge_index[0].reshape(NW * NCHUNK, K)
    dst = edge_index[1].reshape(NW * NCHUNK, K)
    zD = jnp.zeros((NPAD, D1), jnp.float32)
    ones = jnp.ones((K, D1), jnp.float32)
    wih_t = w_ih.T
    whh_t = w_hh.T
    bih2 = b_ih[None, :]
    bhh2 = b_hh[None, :]
    batch3 = batch.reshape(NB, 1, BN)

    m0 = _mm(x, W[0])
    dg = _sc_degree(dst, zD, ones)
    a0 = _sc_aggregate(m0, src, dst, zD)
    h1, m1 = _gru_mid(x, a0[0], a0[1], dg[0], dg[1],
                      wih_t, whh_t, bih2, bhh2, W[1])
    a1 = _sc_aggregate(m1, src, dst, zD)
    pool, cnt = _gru_pool(h1, a1[0], a1[1], dg[0], dg[1],
                          wih_t, whh_t, bih2, bhh2, batch3)
    return _mlp(pool, cnt,
                fc1_w.T, fc1_b[None, :], bn1_g[None, :], bn1_b[None, :],
                fc2_w.T, fc2_b[None, :], bn2_g[None, :], bn2_b[None, :],
                fc3_w.T, fc3_b[None, :])


# speedup vs baseline: 10.0279x; 1.3542x over previous
import functools

import jax
import jax.numpy as jnp
from jax import lax
from jax.experimental import pallas as pl
from jax.experimental.pallas import tpu as pltpu
from jax.experimental.pallas import tpu_sc as plsc

N = 10000
E = 320000
D1 = 128
D2 = 64
D3 = 32
C = 10
G = 64

NC = 2
NS = 16
NW = NC * NS
K = 125
NCHUNK = E // K // NW
EPW = K * NCHUNK
GSZ = 8
NGRP = NCHUNK // GSZ
DDEPTH = 8
NPAD = 10240
RPT = NPAD // NS

BN = 2000
NB = N // BN


def _sc_agg_body(m_hbm, src_hbm, dst_hbm, zD_hbm,
                 out_agg,
                 src_v, dst_v, rows_v, sh_agg,
                 gsem0, gsem1, ssem0, ssem1):
    cid = lax.axis_index("c")
    sid = lax.axis_index("s")
    base_n = sid * RPT
    pltpu.sync_copy(zD_hbm.at[pl.ds(base_n, RPT)],
                    sh_agg.at[pl.ds(base_n, RPT)])
    wid = cid * NS + sid
    pltpu.sync_copy(dst_hbm.at[pl.ds(wid * NCHUNK, NCHUNK)], dst_v)
    pltpu.sync_copy(src_hbm.at[pl.ds(wid * NGRP, 1)],
                    src_v.at[pl.ds(0, 1)])
    plsc.subcore_barrier()

    gsems = (gsem0, gsem1)
    ssems = (ssem0, ssem1)
    for b in range(2):
        pltpu.async_copy(m_hbm.at[src_v.at[0].at[b]], rows_v.at[b],
                         gsems[b])

    def group(j, carry):
        jm = j % 2
        nxt = (j + 1) % 2
        cur_g = src_v.at[pl.ds(jm, 1)].at[0]
        nxt_g = src_v.at[pl.ds(nxt, 1)].at[0]

        @pl.when(j + 1 < NGRP)
        def _load_next_src():
            pltpu.async_copy(
                src_hbm.at[pl.ds(wid * NGRP + j + 1, 1)],
                src_v.at[pl.ds(nxt, 1)], ssems[0])

        for t in range(GSZ):
            i = j * GSZ + t
            pltpu.make_async_copy(m_hbm.at[src_v.at[0].at[0]],
                                  rows_v.at[t % 2], gsems[t % 2]).wait()
            pltpu.sync_copy(rows_v.at[t % 2], sh_agg.at[dst_v.at[i]],
                            add=True)
            if t == GSZ - 2:
                @pl.when(j + 1 < NGRP)
                def _wait_next_src():
                    pltpu.make_async_copy(
                        src_hbm.at[pl.ds(0, 1)],
                        src_v.at[pl.ds(nxt, 1)], ssems[0]).wait()

            @pl.when(i + 2 < NCHUNK)
            def _prefetch():
                if t < GSZ - 2:
                    idx = cur_g.at[t + 2]
                else:
                    idx = nxt_g.at[t + 2 - GSZ]
                pltpu.async_copy(m_hbm.at[idx], rows_v.at[t % 2],
                                 gsems[t % 2])
        return carry

    lax.fori_loop(0, NGRP, group, 0)
    plsc.subcore_barrier()
    base_o = cid * NPAD + base_n
    pltpu.sync_copy(sh_agg.at[pl.ds(base_n, RPT)],
                    out_agg.at[pl.ds(base_o, RPT)])


@functools.lru_cache(maxsize=None)
def _sc_agg():
    mesh = plsc.VectorSubcoreMesh(core_axis_name="c", subcore_axis_name="s")
    return pl.kernel(
        _sc_agg_body,
        mesh=mesh,
        out_type=jax.ShapeDtypeStruct((NC * NPAD, D1), jnp.float32),
        scratch_types=[
            pltpu.VMEM((2, GSZ, K), jnp.int32),
            pltpu.VMEM((NCHUNK, K), jnp.int32),
            pltpu.VMEM((2, K, D1), jnp.float32),
            pltpu.VMEM_SHARED((NPAD, D1), jnp.float32),
            pltpu.SemaphoreType.DMA,
            pltpu.SemaphoreType.DMA,
            pltpu.SemaphoreType.DMA,
            pltpu.SemaphoreType.DMA,
        ],
    )


def _sc_aggregate(m, src2, dst2, zD):
    return _sc_agg()(m, src2, dst2, zD).reshape(NC, NPAD, D1)


def _sc_deg_body(dst_hbm, zD_hbm, ones_hbm,
                 out_deg,
                 dst_v, ones_v, sh_deg, sem):
    cid = lax.axis_index("c")
    sid = lax.axis_index("s")
    base_n = sid * RPT
    pltpu.sync_copy(zD_hbm.at[pl.ds(base_n, RPT)],
                    sh_deg.at[pl.ds(base_n, RPT)])
    pltpu.sync_copy(ones_hbm, ones_v)
    wid = cid * NS + sid
    pltpu.sync_copy(dst_hbm.at[pl.ds(wid * NCHUNK, NCHUNK)], dst_v)
    plsc.subcore_barrier()

    def chunk(i, carry):
        pltpu.async_copy(ones_v, sh_deg.at[dst_v.at[i]], sem, add=True)

        @pl.when(i >= DDEPTH)
        def _drain_one():
            pltpu.make_async_copy(ones_v, sh_deg.at[dst_v.at[0]],
                                  sem).wait()
        return carry

    lax.fori_loop(0, NCHUNK, chunk, 0)

    def drain(i, carry):
        pltpu.make_async_copy(ones_v, sh_deg.at[dst_v.at[0]], sem).wait()
        return carry

    lax.fori_loop(0, DDEPTH, drain, 0)
    plsc.subcore_barrier()
    base_o = cid * NPAD + base_n
    pltpu.sync_copy(sh_deg.at[pl.ds(base_n, RPT)],
                    out_deg.at[pl.ds(base_o, RPT)])


@functools.lru_cache(maxsize=None)
def _sc_deg():
    mesh = plsc.VectorSubcoreMesh(core_axis_name="c", subcore_axis_name="s")
    return pl.kernel(
        _sc_deg_body,
        mesh=mesh,
        out_type=jax.ShapeDtypeStruct((NC * NPAD, D1), jnp.float32),
        scratch_types=[
            pltpu.VMEM((NCHUNK, K), jnp.int32),
            pltpu.VMEM((K, D1), jnp.float32),
            pltpu.VMEM_SHARED((NPAD, D1), jnp.float32),
            pltpu.SemaphoreType.DMA,
        ],
    )


def _sc_degree(dst2, zD, ones):
    return _sc_deg()(dst2, zD, ones).reshape(NC, NPAD, D1)


def _mm_body(h_ref, w_ref, o_ref):
    o_ref[...] = jnp.dot(h_ref[...], w_ref[...],
                         preferred_element_type=jnp.float32)


def _mm(h, w):
    return pl.pallas_call(
        _mm_body,
        grid=(NB,),
        in_specs=[pl.BlockSpec((BN, D1), lambda i: (i, 0)),
                  pl.BlockSpec((D1, D1), lambda i: (0, 0))],
        out_specs=pl.BlockSpec((BN, D1), lambda i: (i, 0)),
        out_shape=jax.ShapeDtypeStruct((N, D1), jnp.float32),
    )(h, w)


def _gru_math(h, p0, p1, d0, d1, wih, whh, bih, bhh):
    deg = jnp.maximum(d0[:, :1] + d1[:, :1], 1.0)
    agg = (p0 + p1) / deg
    gi = jnp.dot(agg, wih, preferred_element_type=jnp.float32) + bih
    gh = jnp.dot(h, whh, preferred_element_type=jnp.float32) + bhh
    r = jax.nn.sigmoid(gi[:, :D1] + gh[:, :D1])
    z = jax.nn.sigmoid(gi[:, D1:2 * D1] + gh[:, D1:2 * D1])
    n = jnp.tanh(gi[:, 2 * D1:] + r * gh[:, 2 * D1:])
    return (1.0 - z) * n + z * h


def _gru_mid_body(h_ref, p0_ref, p1_ref, d0_ref, d1_ref,
                  wih_ref, whh_ref, bih_ref, bhh_ref, w2_ref,
                  h1_ref, m1_ref):
    h1 = _gru_math(h_ref[...], p0_ref[...], p1_ref[...],
                   d0_ref[...], d1_ref[...],
                   wih_ref[...], whh_ref[...], bih_ref[...], bhh_ref[...])
    h1_ref[...] = h1
    m1_ref[...] = jnp.dot(h1, w2_ref[...], preferred_element_type=jnp.float32)


def _gru_mid(h, p0, p1, d0, d1, wih, whh, bih, bhh, w2):
    return pl.pallas_call(
        _gru_mid_body,
        grid=(NB,),
        in_specs=[
            pl.BlockSpec((BN, D1), lambda i: (i, 0)),
            pl.BlockSpec((BN, D1), lambda i: (i, 0)),
            pl.BlockSpec((BN, D1), lambda i: (i, 0)),
            pl.BlockSpec((BN, D1), lambda i: (i, 0)),
            pl.BlockSpec((BN, D1), lambda i: (i, 0)),
            pl.BlockSpec((D1, 3 * D1), lambda i: (0, 0)),
            pl.BlockSpec((D1, 3 * D1), lambda i: (0, 0)),
            pl.BlockSpec((1, 3 * D1), lambda i: (0, 0)),
            pl.BlockSpec((1, 3 * D1), lambda i: (0, 0)),
            pl.BlockSpec((D1, D1), lambda i: (0, 0)),
        ],
        out_specs=[pl.BlockSpec((BN, D1), lambda i: (i, 0)),
                   pl.BlockSpec((BN, D1), lambda i: (i, 0))],
        out_shape=[jax.ShapeDtypeStruct((N, D1), jnp.float32),
                   jax.ShapeDtypeStruct((N, D1), jnp.float32)],
    )(h, p0, p1, d0, d1, wih, whh, bih, bhh, w2)


def _gru_pool_body(h_ref, p0_ref, p1_ref, d0_ref, d1_ref,
                   wih_ref, whh_ref, bih_ref, bhh_ref, b_ref,
                   pool_ref, cnt_ref):
    h2 = _gru_math(h_ref[...], p0_ref[...], p1_ref[...],
                   d0_ref[...], d1_ref[...],
                   wih_ref[...], whh_ref[...], bih_ref[...], bhh_ref[...])
    hr = jnp.maximum(h2, 0.0)
    b = b_ref[0, 0, :]
    oh = (b[:, None] == lax.broadcasted_iota(jnp.int32, (BN, G), 1)
          ).astype(jnp.float32)
    pool_part = lax.dot_general(oh, hr, (((0,), (0,)), ((), ())),
                                preferred_element_type=jnp.float32)
    cnt_part = jnp.sum(oh, axis=0)[:, None] * jnp.ones((1, D1), jnp.float32)

    @pl.when(pl.program_id(0) == 0)
    def _init():
        pool_ref[...] = jnp.zeros_like(pool_ref)
        cnt_ref[...] = jnp.zeros_like(cnt_ref)

    pool_ref[...] += pool_part
    cnt_ref[...] += cnt_part


def _gru_pool(h, p0, p1, d0, d1, wih, whh, bih, bhh, batch3):
    return pl.pallas_call(
        _gru_pool_body,
        grid=(NB,),
        in_specs=[
            pl.BlockSpec((BN, D1), lambda i: (i, 0)),
            pl.BlockSpec((BN, D1), lambda i: (i, 0)),
            pl.BlockSpec((BN, D1), lambda i: (i, 0)),
            pl.BlockSpec((BN, D1), lambda i: (i, 0)),
            pl.BlockSpec((BN, D1), lambda i: (i, 0)),
            pl.BlockSpec((D1, 3 * D1), lambda i: (0, 0)),
            pl.BlockSpec((D1, 3 * D1), lambda i: (0, 0)),
            pl.BlockSpec((1, 3 * D1), lambda i: (0, 0)),
            pl.BlockSpec((1, 3 * D1), lambda i: (0, 0)),
            pl.BlockSpec((1, 1, BN), lambda i: (i, 0, 0)),
        ],
        out_specs=[pl.BlockSpec((G, D1), lambda i: (0, 0)),
                   pl.BlockSpec((G, D1), lambda i: (0, 0))],
        out_shape=[jax.ShapeDtypeStruct((G, D1), jnp.float32),
                   jax.ShapeDtypeStruct((G, D1), jnp.float32)],
    )(h, p0, p1, d0, d1, wih, whh, bih, bhh, batch3)


def _bn_relu(y, g, b):
    mu = jnp.mean(y, axis=0, keepdims=True)
    var = jnp.mean((y - mu) * (y - mu), axis=0, keepdims=True)
    return jnp.maximum((y - mu) / jnp.sqrt(var + 1e-5) * g + b, 0.0)


def _mlp_body(pool_ref, cnt_ref,
              f1_ref, b1_ref, g1_ref, be1_ref,
              f2_ref, b2_ref, g2_ref, be2_ref,
              f3_ref, b3_ref, o_ref):
    p = pool_ref[...] / jnp.maximum(cnt_ref[...], 1.0)
    y = jnp.dot(p, f1_ref[...], preferred_element_type=jnp.float32) + b1_ref[...]
    y = _bn_relu(y, g1_ref[...], be1_ref[...])
    y = jnp.dot(y, f2_ref[...], preferred_element_type=jnp.float32) + b2_ref[...]
    y = _bn_relu(y, g2_ref[...], be2_ref[...])
    y = jnp.dot(y, f3_ref[...], preferred_element_type=jnp.float32) + b3_ref[...]
    m = jnp.max(y, axis=1, keepdims=True)
    lse = jnp.log(jnp.sum(jnp.exp(y - m), axis=1, keepdims=True)) + m
    o_ref[...] = y - lse


def _mlp(pool, cnt, f1t, b1, g1, be1, f2t, b2, g2, be2, f3t, b3):
    full = lambda i: (0, 0)
    return pl.pallas_call(
        _mlp_body,
        grid=(1,),
        in_specs=[
            pl.BlockSpec((G, D1), full), pl.BlockSpec((G, D1), full),
            pl.BlockSpec((D1, D2), full), pl.BlockSpec((1, D2), full),
            pl.BlockSpec((1, D2), full), pl.BlockSpec((1, D2), full),
            pl.BlockSpec((D2, D3), full), pl.BlockSpec((1, D3), full),
            pl.BlockSpec((1, D3), full), pl.BlockSpec((1, D3), full),
            pl.BlockSpec((D3, C), full), pl.BlockSpec((1, C), full),
        ],
        out_specs=pl.BlockSpec((G, C), full),
        out_shape=jax.ShapeDtypeStruct((G, C), jnp.float32),
    )(pool, cnt, f1t, b1, g1, be1, f2t, b2, g2, be2, f3t, b3)


def kernel(x, edge_index, batch, W, w_ih, w_hh, b_ih, b_hh,
           fc1_w, fc1_b, bn1_g, bn1_b, fc2_w, fc2_b, bn2_g, bn2_b,
           fc3_w, fc3_b):
    src = edge_index[0].reshape(NW * NGRP, GSZ, K)
    dst = edge_index[1].reshape(NW * NCHUNK, K)
    zD = jnp.zeros((NPAD, D1), jnp.float32)
    ones = jnp.ones((K, D1), jnp.float32)
    wih_t = w_ih.T
    whh_t = w_hh.T
    bih2 = b_ih[None, :]
    bhh2 = b_hh[None, :]
    batch3 = batch.reshape(NB, 1, BN)

    m0 = _mm(x, W[0])
    dg = _sc_degree(dst, zD, ones)
    a0 = _sc_aggregate(m0, src, dst, zD)
    h1, m1 = _gru_mid(x, a0[0], a0[1], dg[0], dg[1],
                      wih_t, whh_t, bih2, bhh2, W[1])
    a1 = _sc_aggregate(m1, src, dst, zD)
    pool, cnt = _gru_pool(h1, a1[0], a1[1], dg[0], dg[1],
                          wih_t, whh_t, bih2, bhh2, batch3)
    return _mlp(pool, cnt,
                fc1_w.T, fc1_b[None, :], bn1_g[None, :], bn1_b[None, :],
                fc2_w.T, fc2_b[None, :], bn2_g[None, :], bn2_b[None, :],
                fc3_w.T, fc3_b[None, :])


# speedup vs baseline: 10.3668x; 1.0338x over previous
import functools

import jax
import jax.numpy as jnp
from jax import lax
from jax.experimental import pallas as pl
from jax.experimental.pallas import tpu as pltpu
from jax.experimental.pallas import tpu_sc as plsc

N = 10000
E = 320000
D1 = 128
D2 = 64
D3 = 32
C = 10
G = 64

NC = 2
NS = 16
NW = NC * NS
K = 125
NCHUNK = E // K // NW
EPW = K * NCHUNK
GSZ = 8
NGRP = NCHUNK // GSZ
DDEPTH = 16
NPAD = 10240
RPT = NPAD // NS

BN = 2000
NB = N // BN


def _sc_agg_body(m_hbm, src_hbm, dst_hbm, zD_hbm,
                 out_agg,
                 src_v, dst_v, rows_v, sh_agg,
                 gsem0, gsem1, ssem0, ssem1):
    cid = lax.axis_index("c")
    sid = lax.axis_index("s")
    base_n = sid * RPT
    pltpu.sync_copy(zD_hbm.at[pl.ds(base_n, RPT)],
                    sh_agg.at[pl.ds(base_n, RPT)])
    wid = cid * NS + sid
    pltpu.sync_copy(dst_hbm.at[pl.ds(wid * NCHUNK, NCHUNK)], dst_v)
    pltpu.sync_copy(src_hbm.at[pl.ds(wid * NGRP, 1)],
                    src_v.at[pl.ds(0, 1)])
    plsc.subcore_barrier()

    gsems = (gsem0, gsem1)
    ssems = (ssem0, ssem1)
    for b in range(2):
        pltpu.async_copy(m_hbm.at[src_v.at[0].at[b]], rows_v.at[b],
                         gsems[b])

    def group(j, carry):
        jm = j % 2
        nxt = (j + 1) % 2
        cur_g = src_v.at[pl.ds(jm, 1)].at[0]
        nxt_g = src_v.at[pl.ds(nxt, 1)].at[0]

        @pl.when(j + 1 < NGRP)
        def _load_next_src():
            pltpu.async_copy(
                src_hbm.at[pl.ds(wid * NGRP + j + 1, 1)],
                src_v.at[pl.ds(nxt, 1)], ssems[0])

        for t in range(GSZ):
            i = j * GSZ + t
            pltpu.make_async_copy(m_hbm.at[src_v.at[0].at[0]],
                                  rows_v.at[t % 2], gsems[t % 2]).wait()
            pltpu.sync_copy(rows_v.at[t % 2], sh_agg.at[dst_v.at[i]],
                            add=True)
            if t == GSZ - 2:
                @pl.when(j + 1 < NGRP)
                def _wait_next_src():
                    pltpu.make_async_copy(
                        src_hbm.at[pl.ds(0, 1)],
                        src_v.at[pl.ds(nxt, 1)], ssems[0]).wait()

            @pl.when(i + 2 < NCHUNK)
            def _prefetch():
                if t < GSZ - 2:
                    idx = cur_g.at[t + 2]
                else:
                    idx = nxt_g.at[t + 2 - GSZ]
                pltpu.async_copy(m_hbm.at[idx], rows_v.at[t % 2],
                                 gsems[t % 2])
        return carry

    lax.fori_loop(0, NGRP, group, 0)
    plsc.subcore_barrier()
    base_o = cid * NPAD + base_n
    pltpu.sync_copy(sh_agg.at[pl.ds(base_n, RPT)],
                    out_agg.at[pl.ds(base_o, RPT)])


@functools.lru_cache(maxsize=None)
def _sc_agg():
    mesh = plsc.VectorSubcoreMesh(core_axis_name="c", subcore_axis_name="s")
    return pl.kernel(
        _sc_agg_body,
        mesh=mesh,
        out_type=jax.ShapeDtypeStruct((NC * NPAD, D1), jnp.float32),
        scratch_types=[
            pltpu.VMEM((2, GSZ, K), jnp.int32),
            pltpu.VMEM((NCHUNK, K), jnp.int32),
            pltpu.VMEM((2, K, D1), jnp.float32),
            pltpu.VMEM_SHARED((NPAD, D1), jnp.float32),
            pltpu.SemaphoreType.DMA,
            pltpu.SemaphoreType.DMA,
            pltpu.SemaphoreType.DMA,
            pltpu.SemaphoreType.DMA,
        ],
    )


def _sc_aggregate(m, src2, dst2, zD):
    return _sc_agg()(m, src2, dst2, zD).reshape(NC, NPAD, D1)


def _sc_agg_deg_body(m_hbm, src_hbm, dst_hbm, zD_hbm, ones_hbm,
                     out_agg, out_deg,
                     src_v, dst_v, rows_v, sh_agg,
                     gsem0, gsem1, ssem0, ssem1, dsem):
    cid = lax.axis_index("c")
    sid = lax.axis_index("s")
    base_n = sid * RPT
    pltpu.sync_copy(zD_hbm.at[pl.ds(base_n, RPT)],
                    sh_agg.at[pl.ds(base_n, RPT)])
    wid = cid * NS + sid
    pltpu.sync_copy(dst_hbm.at[pl.ds(wid * NCHUNK, NCHUNK)], dst_v)
    pltpu.sync_copy(src_hbm.at[pl.ds(wid * NGRP, 1)],
                    src_v.at[pl.ds(0, 1)])
    plsc.subcore_barrier()

    gsems = (gsem0, gsem1)
    for b in range(2):
        pltpu.async_copy(m_hbm.at[src_v.at[0].at[b]], rows_v.at[b],
                         gsems[b])

    def group(j, carry):
        jm = j % 2
        nxt = (j + 1) % 2
        cur_g = src_v.at[pl.ds(jm, 1)].at[0]
        nxt_g = src_v.at[pl.ds(nxt, 1)].at[0]

        @pl.when(j + 1 < NGRP)
        def _load_next_src():
            pltpu.async_copy(
                src_hbm.at[pl.ds(wid * NGRP + j + 1, 1)],
                src_v.at[pl.ds(nxt, 1)], ssem0)

        for t in range(GSZ):
            i = j * GSZ + t
            pltpu.make_async_copy(m_hbm.at[src_v.at[0].at[0]],
                                  rows_v.at[t % 2], gsems[t % 2]).wait()
            pltpu.sync_copy(rows_v.at[t % 2], sh_agg.at[dst_v.at[i]],
                            add=True)
            if t == GSZ - 2:
                @pl.when(j + 1 < NGRP)
                def _wait_next_src():
                    pltpu.make_async_copy(
                        src_hbm.at[pl.ds(0, 1)],
                        src_v.at[pl.ds(nxt, 1)], ssem0).wait()

            @pl.when(i + 2 < NCHUNK)
            def _prefetch():
                if t < GSZ - 2:
                    idx = cur_g.at[t + 2]
                else:
                    idx = nxt_g.at[t + 2 - GSZ]
                pltpu.async_copy(m_hbm.at[idx], rows_v.at[t % 2],
                                 gsems[t % 2])
        return carry

    lax.fori_loop(0, NGRP, group, 0)
    plsc.subcore_barrier()
    base_o = cid * NPAD + base_n
    pltpu.sync_copy(sh_agg.at[pl.ds(base_n, RPT)],
                    out_agg.at[pl.ds(base_o, RPT)])
    plsc.subcore_barrier()
    pltpu.sync_copy(zD_hbm.at[pl.ds(base_n, RPT)],
                    sh_agg.at[pl.ds(base_n, RPT)])
    ones_v = rows_v.at[0]
    pltpu.sync_copy(ones_hbm, ones_v)
    plsc.subcore_barrier()

    def chunk(i, carry):
        pltpu.async_copy(ones_v, sh_agg.at[dst_v.at[i]], dsem, add=True)

        @pl.when(i >= DDEPTH)
        def _drain_one():
            pltpu.make_async_copy(ones_v, sh_agg.at[dst_v.at[0]],
                                  dsem).wait()
        return carry

    lax.fori_loop(0, NCHUNK, chunk, 0)

    def drain(i, carry):
        pltpu.make_async_copy(ones_v, sh_agg.at[dst_v.at[0]], dsem).wait()
        return carry

    lax.fori_loop(0, DDEPTH, drain, 0)
    plsc.subcore_barrier()
    pltpu.sync_copy(sh_agg.at[pl.ds(base_n, RPT)],
                    out_deg.at[pl.ds(base_o, RPT)])


@functools.lru_cache(maxsize=None)
def _sc_agg_deg():
    mesh = plsc.VectorSubcoreMesh(core_axis_name="c", subcore_axis_name="s")
    return pl.kernel(
        _sc_agg_deg_body,
        mesh=mesh,
        out_type=(jax.ShapeDtypeStruct((NC * NPAD, D1), jnp.float32),
                  jax.ShapeDtypeStruct((NC * NPAD, D1), jnp.float32)),
        scratch_types=[
            pltpu.VMEM((2, GSZ, K), jnp.int32),
            pltpu.VMEM((NCHUNK, K), jnp.int32),
            pltpu.VMEM((2, K, D1), jnp.float32),
            pltpu.VMEM_SHARED((NPAD, D1), jnp.float32),
            pltpu.SemaphoreType.DMA,
            pltpu.SemaphoreType.DMA,
            pltpu.SemaphoreType.DMA,
            pltpu.SemaphoreType.DMA,
            pltpu.SemaphoreType.DMA,
        ],
    )


def _sc_aggregate_deg(m, src3, dst2, zD, ones):
    a, d = _sc_agg_deg()(m, src3, dst2, zD, ones)
    return a.reshape(NC, NPAD, D1), d.reshape(NC, NPAD, D1)


def _gru_math(h, p0, p1, d0, d1, w, wih, whh, bih, bhh):
    deg = jnp.maximum(d0[:, :1] + d1[:, :1], 1.0)
    agg = jnp.dot(p0 + p1, w, preferred_element_type=jnp.float32) / deg
    gi = jnp.dot(agg, wih, preferred_element_type=jnp.float32) + bih
    gh = jnp.dot(h, whh, preferred_element_type=jnp.float32) + bhh
    r = jax.nn.sigmoid(gi[:, :D1] + gh[:, :D1])
    z = jax.nn.sigmoid(gi[:, D1:2 * D1] + gh[:, D1:2 * D1])
    n = jnp.tanh(gi[:, 2 * D1:] + r * gh[:, 2 * D1:])
    return (1.0 - z) * n + z * h


def _gru_mid_body(h_ref, p0_ref, p1_ref, d0_ref, d1_ref,
                  w_ref, wih_ref, whh_ref, bih_ref, bhh_ref,
                  h1_ref):
    h1 = _gru_math(h_ref[...], p0_ref[...], p1_ref[...],
                   d0_ref[...], d1_ref[...], w_ref[...],
                   wih_ref[...], whh_ref[...], bih_ref[...], bhh_ref[...])
    h1_ref[...] = h1


def _gru_mid(h, p0, p1, d0, d1, w, wih, whh, bih, bhh):
    return pl.pallas_call(
        _gru_mid_body,
        grid=(NB,),
        in_specs=[
            pl.BlockSpec((BN, D1), lambda i: (i, 0)),
            pl.BlockSpec((BN, D1), lambda i: (i, 0)),
            pl.BlockSpec((BN, D1), lambda i: (i, 0)),
            pl.BlockSpec((BN, D1), lambda i: (i, 0)),
            pl.BlockSpec((BN, D1), lambda i: (i, 0)),
            pl.BlockSpec((D1, D1), lambda i: (0, 0)),
            pl.BlockSpec((D1, 3 * D1), lambda i: (0, 0)),
            pl.BlockSpec((D1, 3 * D1), lambda i: (0, 0)),
            pl.BlockSpec((1, 3 * D1), lambda i: (0, 0)),
            pl.BlockSpec((1, 3 * D1), lambda i: (0, 0)),
        ],
        out_specs=pl.BlockSpec((BN, D1), lambda i: (i, 0)),
        out_shape=jax.ShapeDtypeStruct((N, D1), jnp.float32),
    )(h, p0, p1, d0, d1, w, wih, whh, bih, bhh)


def _gru_pool_body(h_ref, p0_ref, p1_ref, d0_ref, d1_ref,
                   w_ref, wih_ref, whh_ref, bih_ref, bhh_ref, b_ref,
                   f1_ref, b1_ref, g1_ref, be1_ref,
                   f2_ref, b2_ref, g2_ref, be2_ref,
                   f3_ref, b3_ref,
                   o_ref, pool_ref, cnt_ref):
    h2 = _gru_math(h_ref[...], p0_ref[...], p1_ref[...],
                   d0_ref[...], d1_ref[...], w_ref[...],
                   wih_ref[...], whh_ref[...], bih_ref[...], bhh_ref[...])
    hr = jnp.maximum(h2, 0.0)
    b = b_ref[0, 0, :]
    oh = (b[:, None] == lax.broadcasted_iota(jnp.int32, (BN, G), 1)
          ).astype(jnp.float32)
    pool_part = lax.dot_general(oh, hr, (((0,), (0,)), ((), ())),
                                preferred_element_type=jnp.float32)
    cnt_part = jnp.sum(oh, axis=0)[:, None] * jnp.ones((1, D1), jnp.float32)

    @pl.when(pl.program_id(0) == 0)
    def _init():
        pool_ref[...] = jnp.zeros_like(pool_ref)
        cnt_ref[...] = jnp.zeros_like(cnt_ref)

    pool_ref[...] += pool_part
    cnt_ref[...] += cnt_part

    @pl.when(pl.program_id(0) == NB - 1)
    def _mlp_tail():
        p = pool_ref[...] / jnp.maximum(cnt_ref[...], 1.0)
        y = jnp.dot(p, f1_ref[...],
                    preferred_element_type=jnp.float32) + b1_ref[...]
        y = _bn_relu(y, g1_ref[...], be1_ref[...])
        y = jnp.dot(y, f2_ref[...],
                    preferred_element_type=jnp.float32) + b2_ref[...]
        y = _bn_relu(y, g2_ref[...], be2_ref[...])
        y = jnp.dot(y, f3_ref[...],
                    preferred_element_type=jnp.float32) + b3_ref[...]
        mx = jnp.max(y, axis=1, keepdims=True)
        lse = jnp.log(jnp.sum(jnp.exp(y - mx), axis=1, keepdims=True)) + mx
        o_ref[...] = y - lse


def _gru_pool(h, p0, p1, d0, d1, w, wih, whh, bih, bhh, batch3,
              f1t, b1, g1, be1, f2t, b2, g2, be2, f3t, b3):
    full = lambda i: (0, 0)
    return pl.pallas_call(
        _gru_pool_body,
        grid=(NB,),
        in_specs=[
            pl.BlockSpec((BN, D1), lambda i: (i, 0)),
            pl.BlockSpec((BN, D1), lambda i: (i, 0)),
            pl.BlockSpec((BN, D1), lambda i: (i, 0)),
            pl.BlockSpec((BN, D1), lambda i: (i, 0)),
            pl.BlockSpec((BN, D1), lambda i: (i, 0)),
            pl.BlockSpec((D1, D1), full),
            pl.BlockSpec((D1, 3 * D1), full),
            pl.BlockSpec((D1, 3 * D1), full),
            pl.BlockSpec((1, 3 * D1), full),
            pl.BlockSpec((1, 3 * D1), full),
            pl.BlockSpec((1, 1, BN), lambda i: (i, 0, 0)),
            pl.BlockSpec((D1, D2), full), pl.BlockSpec((1, D2), full),
            pl.BlockSpec((1, D2), full), pl.BlockSpec((1, D2), full),
            pl.BlockSpec((D2, D3), full), pl.BlockSpec((1, D3), full),
            pl.BlockSpec((1, D3), full), pl.BlockSpec((1, D3), full),
            pl.BlockSpec((D3, C), full), pl.BlockSpec((1, C), full),
        ],
        out_specs=pl.BlockSpec((G, C), full),
        out_shape=jax.ShapeDtypeStruct((G, C), jnp.float32),
        scratch_shapes=[pltpu.VMEM((G, D1), jnp.float32),
                        pltpu.VMEM((G, D1), jnp.float32)],
    )(h, p0, p1, d0, d1, w, wih, whh, bih, bhh, batch3,
      f1t, b1, g1, be1, f2t, b2, g2, be2, f3t, b3)


def _bn_relu(y, g, b):
    mu = jnp.mean(y, axis=0, keepdims=True)
    var = jnp.mean((y - mu) * (y - mu), axis=0, keepdims=True)
    return jnp.maximum((y - mu) / jnp.sqrt(var + 1e-5) * g + b, 0.0)


def kernel(x, edge_index, batch, W, w_ih, w_hh, b_ih, b_hh,
           fc1_w, fc1_b, bn1_g, bn1_b, fc2_w, fc2_b, bn2_g, bn2_b,
           fc3_w, fc3_b):
    src = edge_index[0].reshape(NW * NGRP, GSZ, K)
    dst = edge_index[1].reshape(NW * NCHUNK, K)
    zD = jnp.zeros((NPAD, D1), jnp.float32)
    ones = jnp.ones((K, D1), jnp.float32)
    wih_t = w_ih.T
    whh_t = w_hh.T
    bih2 = b_ih[None, :]
    bhh2 = b_hh[None, :]
    batch3 = batch.reshape(NB, 1, BN)

    a0, dg = _sc_aggregate_deg(x, src, dst, zD, ones)
    h1 = _gru_mid(x, a0[0], a0[1], dg[0], dg[1],
                  W[0], wih_t, whh_t, bih2, bhh2)
    a1 = _sc_aggregate(h1, src, dst, zD)
    return _gru_pool(h1, a1[0], a1[1], dg[0], dg[1],
                     W[1], wih_t, whh_t, bih2, bhh2, batch3,
                     fc1_w.T, fc1_b[None, :], bn1_g[None, :], bn1_b[None, :],
                     fc2_w.T, fc2_b[None, :], bn2_g[None, :], bn2_b[None, :],
                     fc3_w.T, fc3_b[None, :])
